# baseline (speedup 1.0000x reference)
import dataclasses
import functools

import jax
import jax.numpy as jnp
from jax import lax
from jax.experimental import pallas as pl
from jax.experimental.pallas import tpu as pltpu
from jax.experimental.pallas import tpu_sc as plsc


_B, _H, _W, _C = 4, 224, 224, 192
_CP = 256
_N = _B * _H * _W
_NW = 32
_PER_W = _N // _NW
_G = 32
_CHUNKS = _PER_W // _G


def _index_body(fx_ref, fy_ref, gx_ref, gy_ref,
                ia_ref, ib_ref, ic_ref, id_ref,
                wa_ref, wb_ref, wc_ref, wd_ref):
    m = pl.program_id(0)
    shape = fx_ref.shape
    r = m * shape[0] + lax.broadcasted_iota(jnp.int32, shape, 0)
    b = ((r >= _H).astype(jnp.int32) + (r >= 2 * _H).astype(jnp.int32)
         + (r >= 3 * _H).astype(jnp.int32))

    x = gx_ref[...] + fx_ref[...] / jnp.float32(_H)
    y = gy_ref[...] + fy_ref[...] / jnp.float32(_H)
    px = (x + 1.0) * jnp.float32(_W) / 2.0
    py = (y + 1.0) * jnp.float32(_H) / 2.0

    x0 = jnp.floor(px).astype(jnp.int32)
    y0 = jnp.floor(py).astype(jnp.int32)
    x1 = x0 + 1
    y1 = y0 + 1
    hi = jnp.int32(_H - 1)
    x0 = jnp.clip(x0, 0, hi)
    x1 = jnp.clip(x1, 0, hi)
    y0 = jnp.clip(y0, 0, hi)
    y1 = jnp.clip(y1, 0, hi)

    x0f = x0.astype(jnp.float32)
    x1f = x1.astype(jnp.float32)
    y0f = y0.astype(jnp.float32)
    y1f = y1.astype(jnp.float32)
    wx0 = x1f - px
    wx1 = px - x0f
    wy0 = y1f - py
    wy1 = py - y0f

    base = b * jnp.int32(_H * _W)
    row0 = base + y0 * jnp.int32(_W)
    row1 = base + y1 * jnp.int32(_W)
    ia_ref[...] = row0 + x0
    ib_ref[...] = row1 + x0
    ic_ref[...] = row0 + x1
    id_ref[...] = row1 + x1
    wa_ref[...] = wx0 * wy0
    wb_ref[...] = wx0 * wy1
    wc_ref[...] = wx1 * wy0
    wd_ref[...] = wx1 * wy1


def _compute_indices(fx, fy, gx, gy):
    rows = _B * _H
    blk = 8
    grid = (rows // blk,)
    row_spec = pl.BlockSpec((blk, _W), lambda m: (m, 0))
    i32 = jax.ShapeDtypeStruct((rows, _W), jnp.int32)
    f32 = jax.ShapeDtypeStruct((rows, _W), jnp.float32)
    return pl.pallas_call(
        _index_body,
        grid=grid,
        in_specs=[
            row_spec,
            row_spec,
            pl.BlockSpec((1, _W), lambda m: (0, 0)),
            pl.BlockSpec((blk, 1), lambda m: (m, 0)),
        ],
        out_specs=[row_spec] * 8,
        out_shape=[i32, i32, i32, i32, f32, f32, f32, f32],
    )(fx, fy, gx, gy)


def _repack_body(i_ref, o_ref):
    x = i_ref[0, 0]
    o_ref[:, 0, :] = x[:, 0:128]
    o_ref[:, 1, 0:64] = x[:, 128:192]


def _repack(conv):
    return pl.pallas_call(
        _repack_body,
        grid=(_B, _H),
        in_specs=[pl.BlockSpec((1, 1, _W, _C), lambda b, h: (b, h, 0, 0))],
        out_specs=pl.BlockSpec((_W, 2, 128), lambda b, h: (b * _H + h, 0, 0)),
        out_shape=jax.ShapeDtypeStruct((_N, 2, 128), jnp.float32),
    )(conv)


def _sc_warp_body(table_hbm, ia_hbm, ib_hbm, ic_hbm, id_hbm,
                  wa_hbm, wb_hbm, wc_hbm, wd_hbm, out_hbm,
                  i0_0, i0_1, i0_2, i0_3, i0_4, i0_5, i0_6, i0_7,
                  i1_0, i1_1, i1_2, i1_3, i1_4, i1_5, i1_6, i1_7,
                  g0_0, g0_1, g0_2, g0_3, g1_0, g1_1, g1_2, g1_3,
                  o0_v, o1_v,
                  isem0, isem1, gsem0, gsem1, osem0, osem1):
    wid = lax.axis_index("s") * 2 + lax.axis_index("c")
    wbase = wid * _PER_W
    srcs = (ia_hbm, ib_hbm, ic_hbm, id_hbm, wa_hbm, wb_hbm, wc_hbm, wd_hbm)
    ibufs = ((i0_0, i0_1, i0_2, i0_3, i0_4, i0_5, i0_6, i0_7),
             (i1_0, i1_1, i1_2, i1_3, i1_4, i1_5, i1_6, i1_7))
    gbufs = ((g0_0, g0_1, g0_2, g0_3), (g1_0, g1_1, g1_2, g1_3))
    obufs = (o0_v, o1_v)
    isems = (isem0, isem1)
    gsems = (gsem0, gsem1)
    osems = (osem0, osem1)
    last = jnp.int32(_CHUNKS - 1)

    def issue_idx(t, s):
        sl = pl.ds(wbase + t * _G, _G)
        for src, dst in zip(srcs, ibufs[s]):
            pltpu.async_copy(src.at[sl], dst, isems[s])

    def wait_idx(s):
        for src, dst in zip(srcs, ibufs[s]):
            pltpu.make_async_copy(src.at[pl.ds(0, _G)], dst, isems[s]).wait()

    def issue_gather(s):
        for iv, gv in zip(ibufs[s][:4], gbufs[s]):
            pltpu.async_copy(table_hbm.at[iv], gv, gsems[s])

    def wait_gather(s):
        for iv, gv in zip(ibufs[s][:4], gbufs[s]):
            pltpu.make_async_copy(table_hbm.at[iv], gv, gsems[s]).wait()

    def issue_out(t, s):
        pltpu.async_copy(obufs[s],
                         out_hbm.at[pl.ds((wbase + t * _G) * _C, _G * _C)],
                         osems[s])

    def wait_out(s):
        pltpu.make_async_copy(obufs[s], out_hbm.at[pl.ds(0, _G * _C)],
                              osems[s]).wait()

    def compute(s):
        wv = ibufs[s][4:]
        av, bv, cv, dv = gbufs[s]
        ov = obufs[s]

        @pl.loop(0, _G)
        def _pix(g):
            gvec = jnp.full((16,), g, dtype=jnp.int32)
            wa = plsc.load_gather(wv[0], [gvec])
            wb = plsc.load_gather(wv[1], [gvec])
            wc = plsc.load_gather(wv[2], [gvec])
            wd = plsc.load_gather(wv[3], [gvec])
            gc = g * _C
            for c in range(0, _C, 16):
                h, cc = divmod(c, 128)
                cs = pl.ds(cc, 16)
                ov[pl.ds(gc + c, 16)] = (
                    wa * av[g, h, cs] + wb * bv[g, h, cs]
                    + wc * cv[g, h, cs] + wd * dv[g, h, cs])

    issue_idx(jnp.int32(0), 0)
    wait_idx(0)
    issue_gather(0)
    issue_idx(jnp.int32(1), 1)

    @pl.loop(0, _CHUNKS // 2)
    def _pair(k):
        c0 = 2 * k

        wait_gather(0)
        wait_idx(1)
        issue_gather(1)

        @pl.when(k >= 1)
        def _():
            wait_out(0)
        compute(0)
        issue_out(c0, 0)
        issue_idx(jnp.minimum(c0 + 2, last), 0)

        wait_gather(1)
        wait_idx(0)
        issue_gather(0)

        @pl.when(k >= 1)
        def _():
            wait_out(1)
        compute(1)
        issue_out(c0 + 1, 1)
        issue_idx(jnp.minimum(c0 + 3, last), 1)

    wait_gather(0)
    wait_idx(1)
    wait_out(0)
    wait_out(1)


def _sc_warp(table, ia, ib, ic, id_, wa, wb, wc, wd):
    mesh = plsc.VectorSubcoreMesh(core_axis_name="c", subcore_axis_name="s")
    cp = pltpu.CompilerParams(
        needs_layout_passes=False, use_tc_tiling_on_sc=False)
    ivec = [pltpu.VMEM((_G,), jnp.int32)] * 4 + [pltpu.VMEM((_G,), jnp.float32)] * 4
    kern = pl.kernel(
        _sc_warp_body,
        out_type=jax.ShapeDtypeStruct((_N * _C,), jnp.float32),
        mesh=mesh,
        scratch_types=(
            ivec + ivec
            + [pltpu.VMEM((_G, 2, 128), jnp.float32)] * 8
            + [pltpu.VMEM((_G * _C,), jnp.float32)] * 2
            + [pltpu.SemaphoreType.DMA] * 6
        ),
        compiler_params=cp,
    )
    return kern(table, ia, ib, ic, id_, wa, wb, wc, wd)


@jax.jit
def kernel(conv_input, flow):
    fx = flow[..., 0].reshape(_B * _H, _W)
    fy = flow[..., 1].reshape(_B * _H, _W)
    gx = jnp.linspace(-1.0, 1.0, _W).reshape(1, _W)
    gy = jnp.tile(jnp.linspace(-1.0, 1.0, _H), _B).reshape(_B * _H, 1)

    ia, ib, ic, id_, wa, wb, wc, wd = _compute_indices(fx, fy, gx, gy)

    table = _repack(jax.lax.optimization_barrier(conv_input))
    out = _sc_warp(
        table,
        ia.reshape(_N), ib.reshape(_N), ic.reshape(_N), id_.reshape(_N),
        wa.reshape(_N), wb.reshape(_N), wc.reshape(_N), wd.reshape(_N),
    )
    return out.reshape(_B, _H, _W, _C)


# speedup vs baseline: 1.2501x; 1.2501x over previous
import dataclasses
import functools

import jax
import jax.numpy as jnp
from jax import lax
from jax.experimental import pallas as pl
from jax.experimental.pallas import tpu as pltpu
from jax.experimental.pallas import tpu_sc as plsc


_B, _H, _W, _C = 4, 224, 224, 192
_CP = 256
_N = _B * _H * _W
_NW = 32
_PER_W = _N // _NW
_G = 32
_CHUNKS = _PER_W // _G


def _index_body(fx_ref, fy_ref, gx_ref, gy_ref,
                ia_ref, ib_ref, ic_ref, id_ref,
                wa_ref, wb_ref, wc_ref, wd_ref):
    m = pl.program_id(0)
    shape = fx_ref.shape
    r = m * shape[0] + lax.broadcasted_iota(jnp.int32, shape, 0)
    b = ((r >= _H).astype(jnp.int32) + (r >= 2 * _H).astype(jnp.int32)
         + (r >= 3 * _H).astype(jnp.int32))

    x = gx_ref[...] + fx_ref[...] / jnp.float32(_H)
    y = gy_ref[...] + fy_ref[...] / jnp.float32(_H)
    px = (x + 1.0) * jnp.float32(_W) / 2.0
    py = (y + 1.0) * jnp.float32(_H) / 2.0

    x0 = jnp.floor(px).astype(jnp.int32)
    y0 = jnp.floor(py).astype(jnp.int32)
    x1 = x0 + 1
    y1 = y0 + 1
    hi = jnp.int32(_H - 1)
    x0 = jnp.clip(x0, 0, hi)
    x1 = jnp.clip(x1, 0, hi)
    y0 = jnp.clip(y0, 0, hi)
    y1 = jnp.clip(y1, 0, hi)

    x0f = x0.astype(jnp.float32)
    x1f = x1.astype(jnp.float32)
    y0f = y0.astype(jnp.float32)
    y1f = y1.astype(jnp.float32)
    wx0 = x1f - px
    wx1 = px - x0f
    wy0 = y1f - py
    wy1 = py - y0f

    base = b * jnp.int32(_H * _W)
    row0 = base + y0 * jnp.int32(_W)
    row1 = base + y1 * jnp.int32(_W)
    ia_ref[...] = row0 + x0
    ib_ref[...] = row1 + x0
    ic_ref[...] = row0 + x1
    id_ref[...] = row1 + x1
    wa_ref[...] = wx0 * wy0
    wb_ref[...] = wx0 * wy1
    wc_ref[...] = wx1 * wy0
    wd_ref[...] = wx1 * wy1


def _compute_indices(fx, fy, gx, gy):
    rows = _B * _H
    blk = 8
    grid = (rows // blk,)
    row_spec = pl.BlockSpec((blk, _W), lambda m: (m, 0))
    i32 = jax.ShapeDtypeStruct((rows, _W), jnp.int32)
    f32 = jax.ShapeDtypeStruct((rows, _W), jnp.float32)
    return pl.pallas_call(
        _index_body,
        grid=grid,
        in_specs=[
            row_spec,
            row_spec,
            pl.BlockSpec((1, _W), lambda m: (0, 0)),
            pl.BlockSpec((blk, 1), lambda m: (m, 0)),
        ],
        out_specs=[row_spec] * 8,
        out_shape=[i32, i32, i32, i32, f32, f32, f32, f32],
    )(fx, fy, gx, gy)


def _repack_body(i_ref, o_ref):
    x = i_ref[0, 0]
    o_ref[:, 0, :] = x[:, 0:128]
    o_ref[:, 1, 0:64] = x[:, 128:192]


def _repack(conv):
    return pl.pallas_call(
        _repack_body,
        grid=(_B, _H),
        in_specs=[pl.BlockSpec((1, 1, _W, _C), lambda b, h: (b, h, 0, 0))],
        out_specs=pl.BlockSpec((_W, 2, 128), lambda b, h: (b * _H + h, 0, 0)),
        out_shape=jax.ShapeDtypeStruct((_N, 2, 128), jnp.float32),
    )(conv)


def _unpack_body(i_ref, o_ref):
    x = i_ref[...].reshape(8, _W, 2, 128)
    o_ref[0, :, :, 0:128] = x[:, :, 0, :]
    o_ref[0, :, :, 128:192] = x[:, :, 1, 0:64]


def _unpack(rows):
    return pl.pallas_call(
        _unpack_body,
        grid=(_B, _H // 8),
        in_specs=[pl.BlockSpec((8 * _W, 2, 128), lambda b, h: (b * (_H // 8) + h, 0, 0))],
        out_specs=pl.BlockSpec((1, 8, _W, _C), lambda b, h: (b, h, 0, 0)),
        out_shape=jax.ShapeDtypeStruct((_B, _H, _W, _C), jnp.float32),
    )(rows)


def _sc_warp_body(table_hbm, ia_hbm, ib_hbm, ic_hbm, id_hbm,
                  wa_hbm, wb_hbm, wc_hbm, wd_hbm, out_hbm,
                  i0_0, i0_1, i0_2, i0_3, i0_4, i0_5, i0_6, i0_7,
                  i1_0, i1_1, i1_2, i1_3, i1_4, i1_5, i1_6, i1_7,
                  g0_0, g0_1, g0_2, g0_3, g1_0, g1_1, g1_2, g1_3,
                  o0_v, o1_v,
                  isem0, isem1, gsem0, gsem1, osem0, osem1):
    wid = lax.axis_index("s") * 2 + lax.axis_index("c")
    wbase = wid * _PER_W
    srcs = (ia_hbm, ib_hbm, ic_hbm, id_hbm, wa_hbm, wb_hbm, wc_hbm, wd_hbm)
    ibufs = ((i0_0, i0_1, i0_2, i0_3, i0_4, i0_5, i0_6, i0_7),
             (i1_0, i1_1, i1_2, i1_3, i1_4, i1_5, i1_6, i1_7))
    gbufs = ((g0_0, g0_1, g0_2, g0_3), (g1_0, g1_1, g1_2, g1_3))
    obufs = (o0_v, o1_v)
    isems = (isem0, isem1)
    gsems = (gsem0, gsem1)
    osems = (osem0, osem1)
    last = jnp.int32(_CHUNKS - 1)

    def issue_idx(t, s):
        sl = pl.ds(wbase + t * _G, _G)
        for src, dst in zip(srcs, ibufs[s]):
            pltpu.async_copy(src.at[sl], dst, isems[s])

    def wait_idx(s):
        for src, dst in zip(srcs, ibufs[s]):
            pltpu.make_async_copy(src.at[pl.ds(0, _G)], dst, isems[s]).wait()

    def issue_gather(s):
        for iv, gv in zip(ibufs[s][:4], gbufs[s]):
            pltpu.async_copy(table_hbm.at[iv], gv, gsems[s])

    def wait_gather(s):
        for iv, gv in zip(ibufs[s][:4], gbufs[s]):
            pltpu.make_async_copy(table_hbm.at[iv], gv, gsems[s]).wait()

    def issue_out(t, s):
        pltpu.async_copy(obufs[s],
                         out_hbm.at[pl.ds(wbase + t * _G, _G)],
                         osems[s])

    def wait_out(s):
        pltpu.make_async_copy(obufs[s], out_hbm.at[pl.ds(0, _G)],
                              osems[s]).wait()

    def compute(s):
        wv = ibufs[s][4:]
        av, bv, cv, dv = gbufs[s]
        ov = obufs[s]

        @pl.loop(0, _G)
        def _pix(g):
            gvec = jnp.full((16,), g, dtype=jnp.int32)
            wa = plsc.load_gather(wv[0], [gvec])
            wb = plsc.load_gather(wv[1], [gvec])
            wc = plsc.load_gather(wv[2], [gvec])
            wd = plsc.load_gather(wv[3], [gvec])
            for c in range(0, _C, 16):
                h, cc = divmod(c, 128)
                cs = pl.ds(cc, 16)
                ov[g, h, cs] = (
                    wa * av[g, h, cs] + wb * bv[g, h, cs]
                    + wc * cv[g, h, cs] + wd * dv[g, h, cs])

    issue_idx(jnp.int32(0), 0)
    wait_idx(0)
    issue_gather(0)
    issue_idx(jnp.int32(1), 1)

    @pl.loop(0, _CHUNKS // 2)
    def _pair(k):
        c0 = 2 * k

        wait_gather(0)
        wait_idx(1)
        issue_gather(1)

        @pl.when(k >= 1)
        def _():
            wait_out(0)
        compute(0)
        issue_out(c0, 0)
        issue_idx(jnp.minimum(c0 + 2, last), 0)

        wait_gather(1)
        wait_idx(0)
        issue_gather(0)

        @pl.when(k >= 1)
        def _():
            wait_out(1)
        compute(1)
        issue_out(c0 + 1, 1)
        issue_idx(jnp.minimum(c0 + 3, last), 1)

    wait_gather(0)
    wait_idx(1)
    wait_out(0)
    wait_out(1)


def _sc_warp(table, ia, ib, ic, id_, wa, wb, wc, wd):
    mesh = plsc.VectorSubcoreMesh(core_axis_name="c", subcore_axis_name="s")
    cp = pltpu.CompilerParams(
        needs_layout_passes=False, use_tc_tiling_on_sc=False)
    ivec = [pltpu.VMEM((_G,), jnp.int32)] * 4 + [pltpu.VMEM((_G,), jnp.float32)] * 4
    kern = pl.kernel(
        _sc_warp_body,
        out_type=jax.ShapeDtypeStruct((_N, 2, 128), jnp.float32),
        mesh=mesh,
        scratch_types=(
            ivec + ivec
            + [pltpu.VMEM((_G, 2, 128), jnp.float32)] * 8
            + [pltpu.VMEM((_G, 2, 128), jnp.float32)] * 2
            + [pltpu.SemaphoreType.DMA] * 6
        ),
        compiler_params=cp,
    )
    return kern(table, ia, ib, ic, id_, wa, wb, wc, wd)


@jax.jit
def kernel(conv_input, flow):
    fx = flow[..., 0].reshape(_B * _H, _W)
    fy = flow[..., 1].reshape(_B * _H, _W)
    gx = jnp.linspace(-1.0, 1.0, _W).reshape(1, _W)
    gy = jnp.tile(jnp.linspace(-1.0, 1.0, _H), _B).reshape(_B * _H, 1)

    ia, ib, ic, id_, wa, wb, wc, wd = _compute_indices(fx, fy, gx, gy)

    table = _repack(jax.lax.optimization_barrier(conv_input))
    out = _sc_warp(
        table,
        ia.reshape(_N), ib.reshape(_N), ic.reshape(_N), id_.reshape(_N),
        wa.reshape(_N), wb.reshape(_N), wc.reshape(_N), wd.reshape(_N),
    )
    return _unpack(out)


# speedup vs baseline: 1.6083x; 1.2866x over previous
import dataclasses
import functools

import jax
import jax.numpy as jnp
from jax import lax
from jax.experimental import pallas as pl
from jax.experimental.pallas import tpu as pltpu
from jax.experimental.pallas import tpu_sc as plsc


_B, _H, _W, _C = 4, 224, 224, 192
_CP = 256
_N = _B * _H * _W
_NW = 32
_PER_W = _N // _NW
_G = 32
_CHUNKS = _PER_W // _G


def _index_body(fx_ref, fy_ref, gx_ref, gy_ref,
                ia_ref, ib_ref, ic_ref, id_ref,
                wa_ref, wb_ref, wc_ref, wd_ref):
    m = pl.program_id(0)
    shape = fx_ref.shape
    r = m * shape[0] + lax.broadcasted_iota(jnp.int32, shape, 0)
    b = ((r >= _H).astype(jnp.int32) + (r >= 2 * _H).astype(jnp.int32)
         + (r >= 3 * _H).astype(jnp.int32))

    x = gx_ref[...] + fx_ref[...] / jnp.float32(_H)
    y = gy_ref[...] + fy_ref[...] / jnp.float32(_H)
    px = (x + 1.0) * jnp.float32(_W) / 2.0
    py = (y + 1.0) * jnp.float32(_H) / 2.0

    x0 = jnp.floor(px).astype(jnp.int32)
    y0 = jnp.floor(py).astype(jnp.int32)
    x1 = x0 + 1
    y1 = y0 + 1
    hi = jnp.int32(_H - 1)
    x0 = jnp.clip(x0, 0, hi)
    x1 = jnp.clip(x1, 0, hi)
    y0 = jnp.clip(y0, 0, hi)
    y1 = jnp.clip(y1, 0, hi)

    x0f = x0.astype(jnp.float32)
    x1f = x1.astype(jnp.float32)
    y0f = y0.astype(jnp.float32)
    y1f = y1.astype(jnp.float32)
    wx0 = x1f - px
    wx1 = px - x0f
    wy0 = y1f - py
    wy1 = py - y0f

    base = b * jnp.int32(_H * _W)
    row0 = base + y0 * jnp.int32(_W)
    row1 = base + y1 * jnp.int32(_W)
    ia_ref[...] = row0 + x0
    ib_ref[...] = row1 + x0
    ic_ref[...] = row0 + x1
    id_ref[...] = row1 + x1
    wa_ref[...] = wx0 * wy0
    wb_ref[...] = wx0 * wy1
    wc_ref[...] = wx1 * wy0
    wd_ref[...] = wx1 * wy1


def _compute_indices(fx, fy, gx, gy):
    rows = _B * _H
    blk = 8
    grid = (rows // blk,)
    row_spec = pl.BlockSpec((blk, _W), lambda m: (m, 0))
    i32 = jax.ShapeDtypeStruct((rows, _W), jnp.int32)
    f32 = jax.ShapeDtypeStruct((rows, _W), jnp.float32)
    return pl.pallas_call(
        _index_body,
        grid=grid,
        in_specs=[
            row_spec,
            row_spec,
            pl.BlockSpec((1, _W), lambda m: (0, 0)),
            pl.BlockSpec((blk, 1), lambda m: (m, 0)),
        ],
        out_specs=[row_spec] * 8,
        out_shape=[i32, i32, i32, i32, f32, f32, f32, f32],
    )(fx, fy, gx, gy)


def _repack_body(i_ref, o_ref):
    x = i_ref[0]
    o_ref[:, 0, :] = x[:, :, 0:128].reshape(16 * _W, 128)
    o_ref[:, 1, 0:64] = x[:, :, 128:192].reshape(16 * _W, 64)


def _repack(conv):
    return pl.pallas_call(
        _repack_body,
        grid=(_B, _H // 16),
        in_specs=[pl.BlockSpec((1, 16, _W, _C), lambda b, h: (b, h, 0, 0))],
        out_specs=pl.BlockSpec((16 * _W, 2, 128),
                               lambda b, h: (b * (_H // 16) + h, 0, 0)),
        out_shape=jax.ShapeDtypeStruct((_N, 2, 128), jnp.float32),
    )(conv)


def _unpack_body(i_ref, o_ref):
    x = i_ref[...].reshape(16, _W, 2, 128)
    o_ref[0, :, :, 0:128] = x[:, :, 0, :]
    o_ref[0, :, :, 128:192] = x[:, :, 1, 0:64]


def _unpack(rows):
    return pl.pallas_call(
        _unpack_body,
        grid=(_B, _H // 16),
        in_specs=[pl.BlockSpec((16 * _W, 2, 128),
                               lambda b, h: (b * (_H // 16) + h, 0, 0))],
        out_specs=pl.BlockSpec((1, 16, _W, _C), lambda b, h: (b, h, 0, 0)),
        out_shape=jax.ShapeDtypeStruct((_B, _H, _W, _C), jnp.float32),
    )(rows)


def _sc_warp_body(table_hbm, ia_hbm, ib_hbm, ic_hbm, id_hbm,
                  wa_hbm, wb_hbm, wc_hbm, wd_hbm, out_hbm,
                  i0_0, i0_1, i0_2, i0_3, i0_4, i0_5, i0_6, i0_7,
                  i1_0, i1_1, i1_2, i1_3, i1_4, i1_5, i1_6, i1_7,
                  g0_0, g0_1, g0_2, g0_3, g1_0, g1_1, g1_2, g1_3,
                  o0_v, o1_v,
                  isem0, isem1, gsem0, gsem1, osem0, osem1):
    wid = lax.axis_index("s") * 2 + lax.axis_index("c")
    wbase = wid * _PER_W
    srcs = (ia_hbm, ib_hbm, ic_hbm, id_hbm, wa_hbm, wb_hbm, wc_hbm, wd_hbm)
    ibufs = ((i0_0, i0_1, i0_2, i0_3, i0_4, i0_5, i0_6, i0_7),
             (i1_0, i1_1, i1_2, i1_3, i1_4, i1_5, i1_6, i1_7))
    gbufs = ((g0_0, g0_1, g0_2, g0_3), (g1_0, g1_1, g1_2, g1_3))
    obufs = (o0_v, o1_v)
    isems = (isem0, isem1)
    gsems = (gsem0, gsem1)
    osems = (osem0, osem1)
    last = jnp.int32(_CHUNKS - 1)

    def issue_idx(t, s):
        sl = pl.ds(wbase + t * _G, _G)
        for src, dst in zip(srcs, ibufs[s]):
            pltpu.async_copy(src.at[sl], dst, isems[s])

    def wait_idx(s):
        for src, dst in zip(srcs, ibufs[s]):
            pltpu.make_async_copy(src.at[pl.ds(0, _G)], dst, isems[s]).wait()

    def issue_gather(s):
        for iv, gv in zip(ibufs[s][:4], gbufs[s]):
            pltpu.async_copy(table_hbm.at[iv], gv, gsems[s])

    def wait_gather(s):
        for iv, gv in zip(ibufs[s][:4], gbufs[s]):
            pltpu.make_async_copy(table_hbm.at[iv], gv, gsems[s]).wait()

    def issue_out(t, s):
        pltpu.async_copy(obufs[s],
                         out_hbm.at[pl.ds(wbase + t * _G, _G)],
                         osems[s])

    def wait_out(s):
        pltpu.make_async_copy(obufs[s], out_hbm.at[pl.ds(0, _G)],
                              osems[s]).wait()

    def compute(s):
        wv = ibufs[s][4:]
        av, bv, cv, dv = gbufs[s]
        ov = obufs[s]

        @pl.loop(0, _G)
        def _pix(g):
            gvec = jnp.full((16,), g, dtype=jnp.int32)
            wa = plsc.load_gather(wv[0], [gvec])
            wb = plsc.load_gather(wv[1], [gvec])
            wc = plsc.load_gather(wv[2], [gvec])
            wd = plsc.load_gather(wv[3], [gvec])
            for c in range(0, _C, 16):
                h, cc = divmod(c, 128)
                cs = pl.ds(cc, 16)
                ov[g, h, cs] = (
                    wa * av[g, h, cs] + wb * bv[g, h, cs]
                    + wc * cv[g, h, cs] + wd * dv[g, h, cs])

    issue_idx(jnp.int32(0), 0)
    wait_idx(0)
    issue_gather(0)
    issue_idx(jnp.int32(1), 1)

    @pl.loop(0, _CHUNKS // 2)
    def _pair(k):
        c0 = 2 * k

        wait_gather(0)
        wait_idx(1)
        issue_gather(1)

        @pl.when(k >= 1)
        def _():
            wait_out(0)
        compute(0)
        issue_out(c0, 0)
        issue_idx(jnp.minimum(c0 + 2, last), 0)

        wait_gather(1)
        wait_idx(0)
        issue_gather(0)

        @pl.when(k >= 1)
        def _():
            wait_out(1)
        compute(1)
        issue_out(c0 + 1, 1)
        issue_idx(jnp.minimum(c0 + 3, last), 1)

    wait_gather(0)
    wait_idx(1)
    wait_out(0)
    wait_out(1)


def _sc_warp(table, ia, ib, ic, id_, wa, wb, wc, wd):
    mesh = plsc.VectorSubcoreMesh(core_axis_name="c", subcore_axis_name="s")
    cp = pltpu.CompilerParams(
        needs_layout_passes=False, use_tc_tiling_on_sc=False)
    ivec = [pltpu.VMEM((_G,), jnp.int32)] * 4 + [pltpu.VMEM((_G,), jnp.float32)] * 4
    kern = pl.kernel(
        _sc_warp_body,
        out_type=jax.ShapeDtypeStruct((_N, 2, 128), jnp.float32),
        mesh=mesh,
        scratch_types=(
            ivec + ivec
            + [pltpu.VMEM((_G, 2, 128), jnp.float32)] * 8
            + [pltpu.VMEM((_G, 2, 128), jnp.float32)] * 2
            + [pltpu.SemaphoreType.DMA] * 6
        ),
        compiler_params=cp,
    )
    return kern(table, ia, ib, ic, id_, wa, wb, wc, wd)


@jax.jit
def kernel(conv_input, flow):
    fx = flow[..., 0].reshape(_B * _H, _W)
    fy = flow[..., 1].reshape(_B * _H, _W)
    gx = jnp.linspace(-1.0, 1.0, _W).reshape(1, _W)
    gy = jnp.tile(jnp.linspace(-1.0, 1.0, _H), _B).reshape(_B * _H, 1)

    ia, ib, ic, id_, wa, wb, wc, wd = _compute_indices(fx, fy, gx, gy)

    table = _repack(jax.lax.optimization_barrier(conv_input))
    out = _sc_warp(
        table,
        ia.reshape(_N), ib.reshape(_N), ic.reshape(_N), id_.reshape(_N),
        wa.reshape(_N), wb.reshape(_N), wc.reshape(_N), wd.reshape(_N),
    )
    return _unpack(out)


# speedup vs baseline: 1.6343x; 1.0162x over previous
import dataclasses
import functools

import jax
import jax.numpy as jnp
from jax import lax
from jax.experimental import pallas as pl
from jax.experimental.pallas import tpu as pltpu
from jax.experimental.pallas import tpu_sc as plsc


_B, _H, _W, _C = 4, 224, 224, 192
_CP = 256
_N = _B * _H * _W
_NW = 32
_PER_W = _N // _NW
_G = 32
_CHUNKS = _PER_W // _G


def _index_body(fx_ref, fy_ref, gx_ref, gy_ref,
                ia_ref, ib_ref, ic_ref, id_ref,
                wa_ref, wb_ref, wc_ref, wd_ref):
    m = pl.program_id(0)
    shape = fx_ref.shape
    r = m * shape[0] + lax.broadcasted_iota(jnp.int32, shape, 0)
    b = ((r >= _H).astype(jnp.int32) + (r >= 2 * _H).astype(jnp.int32)
         + (r >= 3 * _H).astype(jnp.int32))

    x = gx_ref[...] + fx_ref[...] / jnp.float32(_H)
    y = gy_ref[...] + fy_ref[...] / jnp.float32(_H)
    px = (x + 1.0) * jnp.float32(_W) / 2.0
    py = (y + 1.0) * jnp.float32(_H) / 2.0

    x0 = jnp.floor(px).astype(jnp.int32)
    y0 = jnp.floor(py).astype(jnp.int32)
    x1 = x0 + 1
    y1 = y0 + 1
    hi = jnp.int32(_H - 1)
    x0 = jnp.clip(x0, 0, hi)
    x1 = jnp.clip(x1, 0, hi)
    y0 = jnp.clip(y0, 0, hi)
    y1 = jnp.clip(y1, 0, hi)

    x0f = x0.astype(jnp.float32)
    x1f = x1.astype(jnp.float32)
    y0f = y0.astype(jnp.float32)
    y1f = y1.astype(jnp.float32)
    wx0 = x1f - px
    wx1 = px - x0f
    wy0 = y1f - py
    wy1 = py - y0f

    base = b * jnp.int32(_H * _W)
    row0 = base + y0 * jnp.int32(_W)
    row1 = base + y1 * jnp.int32(_W)
    ia_ref[...] = row0 + x0
    ib_ref[...] = row1 + x0
    ic_ref[...] = row0 + x1
    id_ref[...] = row1 + x1
    wa_ref[...] = wx0 * wy0
    wb_ref[...] = wx0 * wy1
    wc_ref[...] = wx1 * wy0
    wd_ref[...] = wx1 * wy1


def _compute_indices(fx, fy, gx, gy):
    rows = _B * _H
    blk = 8
    grid = (rows // blk,)
    row_spec = pl.BlockSpec((blk, _W), lambda m: (m, 0))
    i32 = jax.ShapeDtypeStruct((rows, _W), jnp.int32)
    f32 = jax.ShapeDtypeStruct((rows, _W), jnp.float32)
    return pl.pallas_call(
        _index_body,
        grid=grid,
        in_specs=[
            row_spec,
            row_spec,
            pl.BlockSpec((1, _W), lambda m: (0, 0)),
            pl.BlockSpec((blk, 1), lambda m: (m, 0)),
        ],
        out_specs=[row_spec] * 8,
        out_shape=[i32, i32, i32, i32, f32, f32, f32, f32],
    )(fx, fy, gx, gy)


def _repack_body(i_ref, o_ref):
    x = i_ref[0]
    o_ref[:, 0, :] = x[:, :, 0:128].reshape(28 * _W, 128)
    o_ref[:, 1, 0:64] = x[:, :, 128:192].reshape(28 * _W, 64)


def _repack(conv):
    return pl.pallas_call(
        _repack_body,
        grid=(_B, _H // 28),
        in_specs=[pl.BlockSpec((1, 28, _W, _C), lambda b, h: (b, h, 0, 0))],
        out_specs=pl.BlockSpec((28 * _W, 2, 128),
                               lambda b, h: (b * (_H // 28) + h, 0, 0)),
        out_shape=jax.ShapeDtypeStruct((_N, 2, 128), jnp.float32),
    )(conv)


def _unpack_body(i_ref, o_ref):
    x = i_ref[...].reshape(28, _W, 2, 128)
    o_ref[0, :, :, 0:128] = x[:, :, 0, :]
    o_ref[0, :, :, 128:192] = x[:, :, 1, 0:64]


def _unpack(rows):
    return pl.pallas_call(
        _unpack_body,
        grid=(_B, _H // 28),
        in_specs=[pl.BlockSpec((28 * _W, 2, 128),
                               lambda b, h: (b * (_H // 28) + h, 0, 0))],
        out_specs=pl.BlockSpec((1, 28, _W, _C), lambda b, h: (b, h, 0, 0)),
        out_shape=jax.ShapeDtypeStruct((_B, _H, _W, _C), jnp.float32),
    )(rows)


def _sc_warp_body(table_hbm, ia_hbm, ib_hbm, ic_hbm, id_hbm,
                  wa_hbm, wb_hbm, wc_hbm, wd_hbm, out_hbm,
                  i0_0, i0_1, i0_2, i0_3, i0_4, i0_5, i0_6, i0_7,
                  i1_0, i1_1, i1_2, i1_3, i1_4, i1_5, i1_6, i1_7,
                  g0_0, g0_1, g0_2, g0_3, g1_0, g1_1, g1_2, g1_3,
                  o0_v, o1_v,
                  isem0, isem1, gsem0, gsem1, osem0, osem1):
    wid = lax.axis_index("s") * 2 + lax.axis_index("c")
    wbase = wid * _PER_W
    srcs = (ia_hbm, ib_hbm, ic_hbm, id_hbm, wa_hbm, wb_hbm, wc_hbm, wd_hbm)
    ibufs = ((i0_0, i0_1, i0_2, i0_3, i0_4, i0_5, i0_6, i0_7),
             (i1_0, i1_1, i1_2, i1_3, i1_4, i1_5, i1_6, i1_7))
    gbufs = ((g0_0, g0_1, g0_2, g0_3), (g1_0, g1_1, g1_2, g1_3))
    obufs = (o0_v, o1_v)
    isems = (isem0, isem1)
    gsems = (gsem0, gsem1)
    osems = (osem0, osem1)
    last = jnp.int32(_CHUNKS - 1)

    def issue_idx(t, s):
        sl = pl.ds(wbase + t * _G, _G)
        for src, dst in zip(srcs, ibufs[s]):
            pltpu.async_copy(src.at[sl], dst, isems[s])

    def wait_idx(s):
        for src, dst in zip(srcs, ibufs[s]):
            pltpu.make_async_copy(src.at[pl.ds(0, _G)], dst, isems[s]).wait()

    def issue_gather(s):
        for iv, gv in zip(ibufs[s][:4], gbufs[s]):
            pltpu.async_copy(table_hbm.at[iv], gv, gsems[s])

    def wait_gather(s):
        for iv, gv in zip(ibufs[s][:4], gbufs[s]):
            pltpu.make_async_copy(table_hbm.at[iv], gv, gsems[s]).wait()

    def issue_out(t, s):
        pltpu.async_copy(obufs[s],
                         out_hbm.at[pl.ds(wbase + t * _G, _G)],
                         osems[s])

    def wait_out(s):
        pltpu.make_async_copy(obufs[s], out_hbm.at[pl.ds(0, _G)],
                              osems[s]).wait()

    def compute(s):
        wv = ibufs[s][4:]
        av, bv, cv, dv = gbufs[s]
        ov = obufs[s]

        @pl.loop(0, _G)
        def _pix(g):
            gvec = jnp.full((16,), g, dtype=jnp.int32)
            wa = plsc.load_gather(wv[0], [gvec])
            wb = plsc.load_gather(wv[1], [gvec])
            wc = plsc.load_gather(wv[2], [gvec])
            wd = plsc.load_gather(wv[3], [gvec])
            for c in range(0, _C, 16):
                h, cc = divmod(c, 128)
                cs = pl.ds(cc, 16)
                ov[g, h, cs] = (
                    wa * av[g, h, cs] + wb * bv[g, h, cs]
                    + wc * cv[g, h, cs] + wd * dv[g, h, cs])

    issue_idx(jnp.int32(0), 0)
    wait_idx(0)
    issue_gather(0)
    issue_idx(jnp.int32(1), 1)

    @pl.loop(0, _CHUNKS // 2)
    def _pair(k):
        c0 = 2 * k

        wait_gather(0)
        wait_idx(1)
        issue_gather(1)

        @pl.when(k >= 1)
        def _():
            wait_out(0)
        compute(0)
        issue_out(c0, 0)
        issue_idx(jnp.minimum(c0 + 2, last), 0)

        wait_gather(1)
        wait_idx(0)
        issue_gather(0)

        @pl.when(k >= 1)
        def _():
            wait_out(1)
        compute(1)
        issue_out(c0 + 1, 1)
        issue_idx(jnp.minimum(c0 + 3, last), 1)

    wait_gather(0)
    wait_idx(1)
    wait_out(0)
    wait_out(1)


def _sc_warp(table, ia, ib, ic, id_, wa, wb, wc, wd):
    mesh = plsc.VectorSubcoreMesh(core_axis_name="c", subcore_axis_name="s")
    cp = pltpu.CompilerParams(
        needs_layout_passes=False, use_tc_tiling_on_sc=False)
    ivec = [pltpu.VMEM((_G,), jnp.int32)] * 4 + [pltpu.VMEM((_G,), jnp.float32)] * 4
    kern = pl.kernel(
        _sc_warp_body,
        out_type=jax.ShapeDtypeStruct((_N, 2, 128), jnp.float32),
        mesh=mesh,
        scratch_types=(
            ivec + ivec
            + [pltpu.VMEM((_G, 2, 128), jnp.float32)] * 8
            + [pltpu.VMEM((_G, 2, 128), jnp.float32)] * 2
            + [pltpu.SemaphoreType.DMA] * 6
        ),
        compiler_params=cp,
    )
    return kern(table, ia, ib, ic, id_, wa, wb, wc, wd)


@jax.jit
def kernel(conv_input, flow):
    fx = flow[..., 0].reshape(_B * _H, _W)
    fy = flow[..., 1].reshape(_B * _H, _W)
    gx = jnp.linspace(-1.0, 1.0, _W).reshape(1, _W)
    gy = jnp.tile(jnp.linspace(-1.0, 1.0, _H), _B).reshape(_B * _H, 1)

    ia, ib, ic, id_, wa, wb, wc, wd = _compute_indices(fx, fy, gx, gy)

    table = _repack(jax.lax.optimization_barrier(conv_input))
    out = _sc_warp(
        table,
        ia.reshape(_N), ib.reshape(_N), ic.reshape(_N), id_.reshape(_N),
        wa.reshape(_N), wb.reshape(_N), wc.reshape(_N), wd.reshape(_N),
    )
    return _unpack(out)
